# baseline (speedup 1.0000x reference)
import functools

import jax
import jax.numpy as jnp
from jax import lax
from jax.experimental import pallas as pl
from jax.experimental.pallas import tpu as pltpu
from jax.experimental.pallas import tpu_sc as plsc

_N0, _N1, _N2 = 250000, 20480, 2048
_E0, _E1 = 512000, 20480
_D = 128
_NC, _NS = 2, 16
_CH = 64
_K = 10


def _chunk_plan(total, maxc):
    sizes = [maxc] * (total // maxc)
    if total % maxc:
        sizes.append(total % maxc)
    return sizes


def _make_sc_agg(n_tgt, n_edges):
    half = n_tgt // _NC
    chunks = n_edges // (_NS * _CH)
    groups = chunks // _K
    assert chunks * _NS * _CH == n_edges and half * _NC == n_tgt
    assert groups * _K == chunks
    rows_sh = half + 128
    zper = rows_sh // _NS
    assert zper * _NS == rows_sh and zper % 8 == 0
    wper = half // _NS
    assert wper * _NS == half and wper % 8 == 0
    mesh = plsc.VectorSubcoreMesh(core_axis_name="c", subcore_axis_name="s")

    @functools.partial(
        pl.kernel,
        mesh=mesh,
        compiler_params=pltpu.CompilerParams(
            needs_layout_passes=False, use_tc_tiling_on_sc=False),
        out_type=[
            jax.ShapeDtypeStruct((n_tgt, _D), jnp.float32),
            jax.ShapeDtypeStruct((_NS * n_tgt,), jnp.float32),
        ],
        scratch_types=[
            pltpu.VMEM((_K * _CH,), jnp.int32),
            pltpu.VMEM((_K * _CH,), jnp.int32),
            pltpu.VMEM((_K * _CH + 80,), jnp.int32),
            pltpu.VMEM((_K * _CH + 80,), jnp.int32),
            pltpu.VMEM((_CH,), jnp.int32),
            pltpu.VMEM((_CH,), jnp.int32),
            pltpu.VMEM((_CH, _D), jnp.float32),
            pltpu.VMEM((_CH, _D), jnp.float32),
            pltpu.VMEM((half,), jnp.float32),
            pltpu.VMEM_SHARED((rows_sh, _D), jnp.float32),
            pltpu.SemaphoreType.DMA,
            pltpu.SemaphoreType.DMA,
            pltpu.SemaphoreType.DMA,
            pltpu.SemaphoreType.DMA,
        ],
    )
    def agg_kernel(src_hbm, dst_hbm, x_hbm, agg_hbm, hist_hbm,
                   idx_v, dst_v, psrc_v, ploc_v, sca_v, scb_v,
                   rows_a, rows_b, hist_v, agg_sh,
                   sem_a, sem_b, sem_sa, sem_sb):
        core = lax.axis_index("c")
        sid = lax.axis_index("s")
        core_base = core * half

        zero16 = jnp.zeros((16,), jnp.float32)
        one16 = jnp.ones((16,), jnp.float32)

        def zrows_body(i, _):
            for j in range(_D // 16):
                rows_a[i, pl.ds(j * 16, 16)] = zero16
                rows_b[i, pl.ds(j * 16, 16)] = zero16
            return 0

        lax.fori_loop(0, _CH, zrows_body, 0)

        def zhist_body(i, _):
            hist_v[pl.ds(i * 16, 16)] = zero16
            return 0

        lax.fori_loop(0, half // 16, zhist_body, 0)

        zbase = sid * zper
        off = 0
        for sz in _chunk_plan(zper, _CH):
            pltpu.sync_copy(rows_a.at[pl.ds(0, sz)],
                            agg_sh.at[pl.ds(zbase + off, sz)])
            off += sz
        plsc.subcore_barrier()

        def _fire_gather(k, buf, sem):
            return pltpu.async_copy(
                x_hbm.at[psrc_v.at[pl.ds(k * _CH, _CH)]], buf, sem)

        def _wait_gather(k, buf, sem):
            pltpu.make_async_copy(
                x_hbm.at[psrc_v.at[pl.ds(k * _CH, _CH)]], buf, sem).wait()

        def _wait_scatter(buf, sc_idx, sem):
            pltpu.make_async_copy(buf, agg_sh.at[sc_idx], sem).wait()

        def group_body(g, _):
            base = sid * chunks * _CH + g * (_K * _CH)
            pltpu.sync_copy(src_hbm.at[pl.ds(base, _K * _CH)], idx_v)
            pltpu.sync_copy(dst_hbm.at[pl.ds(base, _K * _CH)], dst_v)

            def munge_body(m, c):
                dv = dst_v[pl.ds(m * 16, 16)]
                sv = idx_v[pl.ds(m * 16, 16)]
                local = dv - core_base
                ok = (local >= 0) & (local < half)
                plsc.addupdate_scatter(hist_v, [jnp.where(ok, local, 0)],
                                       jnp.where(ok, one16, 0.0))
                plsc.store_compressed(psrc_v.at[pl.ds(c, 16)], sv, mask=ok)
                plsc.store_compressed(ploc_v.at[pl.ds(c, 16)], local,
                                      mask=ok)
                return c + jnp.sum(ok.astype(jnp.int32))

            c = lax.fori_loop(0, (_K * _CH) // 16, munge_body,
                              jnp.int32(0))
            for t in range(_CH // 16):
                psrc_v[pl.ds(c + t * 16, 16)] = jnp.zeros((16,), jnp.int32)
                ploc_v[pl.ds(c + t * 16, 16)] = jnp.full((16,), half,
                                                         jnp.int32)
            nfull = (c + (_CH - 1)) // _CH

            @pl.when(nfull > 0)
            def _():
                _fire_gather(0, rows_a, sem_a)

            def pipe_body(k, _):
                even = (k % 2) == 0

                @pl.when((k >= 1) & ~even)
                def _():
                    _wait_scatter(rows_a, sca_v, sem_sa)

                @pl.when((k >= 1) & even)
                def _():
                    _wait_scatter(rows_b, scb_v, sem_sb)

                @pl.when((k + 1 < nfull) & even)
                def _():
                    _fire_gather(k + 1, rows_b, sem_b)

                @pl.when((k + 1 < nfull) & ~even)
                def _():
                    _fire_gather(k + 1, rows_a, sem_a)

                @pl.when(even)
                def _():
                    _wait_gather(k, rows_a, sem_a)
                    for j in range(_CH // 16):
                        sca_v[pl.ds(j * 16, 16)] = (
                            ploc_v[pl.ds(k * _CH + j * 16, 16)])
                    pltpu.async_copy(rows_a, agg_sh.at[sca_v], sem_sa,
                                     add=True)

                @pl.when(~even)
                def _():
                    _wait_gather(k, rows_b, sem_b)
                    for j in range(_CH // 16):
                        scb_v[pl.ds(j * 16, 16)] = (
                            ploc_v[pl.ds(k * _CH + j * 16, 16)])
                    pltpu.async_copy(rows_b, agg_sh.at[scb_v], sem_sb,
                                     add=True)
                return 0

            lax.fori_loop(0, nfull, pipe_body, 0)

            @pl.when((nfull > 0) & ((nfull % 2) == 1))
            def _():
                _wait_scatter(rows_a, sca_v, sem_sa)

            @pl.when((nfull > 0) & ((nfull % 2) == 0))
            def _():
                _wait_scatter(rows_b, scb_v, sem_sb)
            return 0

        lax.fori_loop(0, groups, group_body, 0)
        plsc.subcore_barrier()

        wbase = sid * wper
        off = 0
        for sz in _chunk_plan(wper, _CH):
            pltpu.sync_copy(agg_sh.at[pl.ds(wbase + off, sz)],
                            rows_a.at[pl.ds(0, sz)])
            pltpu.sync_copy(rows_a.at[pl.ds(0, sz)],
                            agg_hbm.at[pl.ds(core_base + wbase + off, sz)])
            off += sz
        pltpu.sync_copy(hist_v,
                        hist_hbm.at[pl.ds(sid * n_tgt + core_base, half)])

    return agg_kernel


def _make_dense(n, relu):
    blk = min(n, 1024)
    assert n % blk == 0

    def body(agg_ref, hist_ref, xt_ref, wl_ref, wr_ref, b_ref, o_ref):
        ones_col = jnp.ones((_NS, 1), jnp.float32)
        cnt = lax.dot_general(hist_ref[...], ones_col,
                              (((0,), (0,)), ((), ())),
                              preferred_element_type=jnp.float32)
        mean = agg_ref[...] / jnp.maximum(cnt, 1.0)
        acc = jnp.dot(mean, wl_ref[...], preferred_element_type=jnp.float32,
                      precision=lax.Precision.HIGHEST)
        acc += jnp.dot(xt_ref[...], wr_ref[...],
                       preferred_element_type=jnp.float32,
                       precision=lax.Precision.HIGHEST)
        acc += b_ref[...]
        o_ref[...] = jnp.maximum(acc, 0.0) if relu else acc

    return pl.pallas_call(
        body,
        grid=(n // blk,),
        in_specs=[
            pl.BlockSpec((blk, _D), lambda i: (i, 0)),
            pl.BlockSpec((_NS, blk), lambda i: (0, i)),
            pl.BlockSpec((blk, _D), lambda i: (i, 0)),
            pl.BlockSpec((_D, _D), lambda i: (0, 0)),
            pl.BlockSpec((_D, _D), lambda i: (0, 0)),
            pl.BlockSpec((1, _D), lambda i: (0, 0)),
        ],
        out_specs=pl.BlockSpec((blk, _D), lambda i: (i, 0)),
        out_shape=jax.ShapeDtypeStruct((n, _D), jnp.float32),
    )


_agg0 = _make_sc_agg(_N1, _E0)
_agg1 = _make_sc_agg(_N2, _E1)
_dense0 = _make_dense(_N1, relu=True)
_dense1 = _make_dense(_N2, relu=False)


def kernel(x, edge_src_0, edge_dst_0, edge_src_1, edge_dst_1,
           W_l0, W_r0, b0, W_l1, W_r1, b1):
    es0 = edge_src_0.astype(jnp.int32)
    ed0 = edge_dst_0.astype(jnp.int32)
    es1 = edge_src_1.astype(jnp.int32)
    ed1 = edge_dst_1.astype(jnp.int32)
    agg0, hist0 = _agg0(es0, ed0, x)
    h = _dense0(agg0, hist0.reshape(_NS, _N1), x[:_N1], W_l0, W_r0,
                b0.reshape(1, _D))
    agg1, hist1 = _agg1(es1, ed1, h)
    out = _dense1(agg1, hist1.reshape(_NS, _N2), h[:_N2], W_l1, W_r1,
                  b1.reshape(1, _D))
    return out


# speedup vs baseline: 3.7449x; 3.7449x over previous
import functools

import jax
import jax.numpy as jnp
from jax import lax
from jax.experimental import pallas as pl
from jax.experimental.pallas import tpu as pltpu
from jax.experimental.pallas import tpu_sc as plsc

_N0, _N1, _N2 = 250000, 20480, 2048
_E0, _E1 = 512000, 20480
_D = 128
_NC, _NS = 2, 16
_CH = 64
_K = 10


def _chunk_plan(total, maxc):
    sizes = [maxc] * (total // maxc)
    if total % maxc:
        sizes.append(total % maxc)
    return sizes


def _make_sc_agg(n_tgt, n_edges):
    half = n_tgt // _NC
    chunks = n_edges // (_NS * _CH)
    groups = chunks // _K
    assert chunks * _NS * _CH == n_edges and half * _NC == n_tgt
    assert groups * _K == chunks
    rows_sh = half + 128
    zper = rows_sh // _NS
    assert zper * _NS == rows_sh and zper % 8 == 0
    wper = half // _NS
    assert wper * _NS == half and wper % 8 == 0
    mesh = plsc.VectorSubcoreMesh(core_axis_name="c", subcore_axis_name="s")

    @functools.partial(
        pl.kernel,
        mesh=mesh,
        compiler_params=pltpu.CompilerParams(
            needs_layout_passes=False, use_tc_tiling_on_sc=False),
        out_type=[
            jax.ShapeDtypeStruct((n_tgt, _D), jnp.float32),
            jax.ShapeDtypeStruct((_NS * n_tgt,), jnp.float32),
        ],
        scratch_types=[
            pltpu.VMEM((_K * _CH,), jnp.int32),
            pltpu.VMEM((_K * _CH,), jnp.int32),
            pltpu.VMEM((_K, _CH), jnp.int32),
            pltpu.VMEM((_CH, _D), jnp.float32),
            pltpu.VMEM((_CH, _D), jnp.float32),
            pltpu.VMEM((half,), jnp.float32),
            pltpu.VMEM_SHARED((rows_sh, _D), jnp.float32),
            pltpu.SemaphoreType.DMA,
            pltpu.SemaphoreType.DMA,
            pltpu.SemaphoreType.DMA,
            pltpu.SemaphoreType.DMA,
        ],
    )
    def agg_kernel(src_hbm, dst_hbm, x_hbm, agg_hbm, hist_hbm,
                   idx_v, dst_v, loc_v, rows_a, rows_b, hist_v, agg_sh,
                   sem_a, sem_b, sem_sa, sem_sb):
        core = lax.axis_index("c")
        sid = lax.axis_index("s")
        core_base = core * half

        zero16 = jnp.zeros((16,), jnp.float32)
        one16 = jnp.ones((16,), jnp.float32)

        def zrows_body(i, _):
            for j in range(_D // 16):
                rows_a[i, pl.ds(j * 16, 16)] = zero16
                rows_b[i, pl.ds(j * 16, 16)] = zero16
            return 0

        lax.fori_loop(0, _CH, zrows_body, 0)

        def zhist_body(i, _):
            hist_v[pl.ds(i * 16, 16)] = zero16
            return 0

        lax.fori_loop(0, half // 16, zhist_body, 0)

        zbase = sid * zper
        off = 0
        for sz in _chunk_plan(zper, _CH):
            pltpu.sync_copy(rows_a.at[pl.ds(0, sz)],
                            agg_sh.at[pl.ds(zbase + off, sz)])
            off += sz
        plsc.subcore_barrier()

        bufs = (rows_a, rows_b)
        sems = (sem_a, sem_b)
        ssems = (sem_sa, sem_sb)

        def group_body(g, _):
            base = sid * chunks * _CH + g * (_K * _CH)
            pltpu.sync_copy(src_hbm.at[pl.ds(base, _K * _CH)], idx_v)
            pltpu.sync_copy(dst_hbm.at[pl.ds(base, _K * _CH)], dst_v)

            def munge_body(m, _):
                dv = dst_v[pl.ds(m * 16, 16)]
                local = dv - core_base
                ok = (local >= 0) & (local < half)
                loc_v[m // (_CH // 16), pl.ds((m % (_CH // 16)) * 16, 16)] = (
                    jnp.where(ok, local, half))
                plsc.addupdate_scatter(hist_v, [jnp.where(ok, local, 0)],
                                       jnp.where(ok, one16, 0.0))
                return 0

            lax.fori_loop(0, (_K * _CH) // 16, munge_body, 0)

            cp = pltpu.async_copy(x_hbm.at[idx_v.at[pl.ds(0, _CH)]],
                                  bufs[0], sems[0])
            sc = [None, None]
            for k in range(_K):
                b = k % 2
                nb = 1 - b
                if sc[nb] is not None:
                    sc[nb].wait()
                    sc[nb] = None
                nxt = None
                if k + 1 < _K:
                    nxt = pltpu.async_copy(
                        x_hbm.at[idx_v.at[pl.ds((k + 1) * _CH, _CH)]],
                        bufs[nb], sems[nb])
                cp.wait()
                sc[b] = pltpu.async_copy(bufs[b], agg_sh.at[loc_v.at[k]],
                                         ssems[b], add=True)
                cp = nxt
            for d in sc:
                if d is not None:
                    d.wait()
            return 0

        lax.fori_loop(0, groups, group_body, 0)
        plsc.subcore_barrier()

        wbase = sid * wper
        off = 0
        for sz in _chunk_plan(wper, _CH):
            pltpu.sync_copy(agg_sh.at[pl.ds(wbase + off, sz)],
                            rows_a.at[pl.ds(0, sz)])
            pltpu.sync_copy(rows_a.at[pl.ds(0, sz)],
                            agg_hbm.at[pl.ds(core_base + wbase + off, sz)])
            off += sz
        pltpu.sync_copy(hist_v,
                        hist_hbm.at[pl.ds(sid * n_tgt + core_base, half)])

    return agg_kernel


def _make_dense(n, relu):
    blk = min(n, 1024)
    assert n % blk == 0

    def body(agg_ref, hist_ref, xt_ref, wl_ref, wr_ref, b_ref, o_ref):
        ones_col = jnp.ones((_NS, 1), jnp.float32)
        cnt = lax.dot_general(hist_ref[...], ones_col,
                              (((0,), (0,)), ((), ())),
                              preferred_element_type=jnp.float32)
        mean = agg_ref[...] / jnp.maximum(cnt, 1.0)
        acc = jnp.dot(mean, wl_ref[...], preferred_element_type=jnp.float32,
                      precision=lax.Precision.HIGHEST)
        acc += jnp.dot(xt_ref[...], wr_ref[...],
                       preferred_element_type=jnp.float32,
                       precision=lax.Precision.HIGHEST)
        acc += b_ref[...]
        o_ref[...] = jnp.maximum(acc, 0.0) if relu else acc

    return pl.pallas_call(
        body,
        grid=(n // blk,),
        in_specs=[
            pl.BlockSpec((blk, _D), lambda i: (i, 0)),
            pl.BlockSpec((_NS, blk), lambda i: (0, i)),
            pl.BlockSpec((blk, _D), lambda i: (i, 0)),
            pl.BlockSpec((_D, _D), lambda i: (0, 0)),
            pl.BlockSpec((_D, _D), lambda i: (0, 0)),
            pl.BlockSpec((1, _D), lambda i: (0, 0)),
        ],
        out_specs=pl.BlockSpec((blk, _D), lambda i: (i, 0)),
        out_shape=jax.ShapeDtypeStruct((n, _D), jnp.float32),
    )


_agg0 = _make_sc_agg(_N1, _E0)
_agg1 = _make_sc_agg(_N2, _E1)
_dense0 = _make_dense(_N1, relu=True)
_dense1 = _make_dense(_N2, relu=False)


def kernel(x, edge_src_0, edge_dst_0, edge_src_1, edge_dst_1,
           W_l0, W_r0, b0, W_l1, W_r1, b1):
    es0 = edge_src_0.astype(jnp.int32)
    ed0 = edge_dst_0.astype(jnp.int32)
    es1 = edge_src_1.astype(jnp.int32)
    ed1 = edge_dst_1.astype(jnp.int32)
    agg0, hist0 = _agg0(es0, ed0, x)
    h = _dense0(agg0, hist0.reshape(_NS, _N1), x[:_N1], W_l0, W_r0,
                b0.reshape(1, _D))
    agg1, hist1 = _agg1(es1, ed1, h)
    out = _dense1(agg1, hist1.reshape(_NS, _N2), h[:_N2], W_l1, W_r1,
                  b1.reshape(1, _D))
    return out
